# baseline (speedup 1.0000x reference)
import functools

import jax
import jax.numpy as jnp
from jax import lax
from jax.experimental import pallas as pl
from jax.experimental.pallas import tpu as pltpu
from jax.experimental.pallas import tpu_sc as plsc

NC = 2
NS = 16
NW = NC * NS

B, S = 16384, 50
D = 64
DP = 128
V = 1_000_000
CHUNK = 128

NFULL = V // CHUNK
K1_PER_W = NFULL // NW
K1_EXTRA = NFULL - K1_PER_W * NW
TAIL_R0 = NFULL * CHUNK
TAIL_N = V - TAIL_R0

BCOLS = B // NW
K2_N = S * (BCOLS // CHUNK)
GB = 4

_i32 = jnp.int32


def _iota16(off):
    return jnp.arange(16, dtype=_i32) + off


def kernel(batch_data, item_embeddings):
    table_t = jnp.transpose(item_embeddings)
    idx_t = jnp.transpose(batch_data)
    mesh1 = plsc.VectorSubcoreMesh(core_axis_name="c", subcore_axis_name="s")
    mesh2 = plsc.VectorSubcoreMesh(core_axis_name="c", subcore_axis_name="s")
    params = pltpu.CompilerParams(
        use_tc_tiling_on_sc=True, needs_layout_passes=False
    )

    @functools.partial(
        pl.kernel,
        mesh=mesh1,
        compiler_params=params,
        out_type=jax.ShapeDtypeStruct((V, DP), jnp.float32),
        scratch_types=[
            pltpu.VMEM((2, D, CHUNK), jnp.float32),
            pltpu.VMEM((2, CHUNK, DP), jnp.float32),
        ] + [pltpu.SemaphoreType.DMA] * 4,
    )
    def retile_kernel(tab_hbm, tail_hbm, out_hbm, tbuf, obuf, rs0, rs1, ws0, ws1):
        rsem = (rs0, rs1)
        wsem = (ws0, ws1)
        wid = lax.axis_index("s") * NC + lax.axis_index("c")
        blk0 = wid * K1_PER_W

        def read_copy(t, p):
            r0 = (blk0 + t) * CHUNK
            return pltpu.make_async_copy(
                tab_hbm.at[:, pl.ds(r0, CHUNK)], tbuf.at[p], rsem[p]
            )

        def write_copy(t, p):
            r0 = (blk0 + t) * CHUNK
            return pltpu.make_async_copy(
                obuf.at[p], out_hbm.at[pl.ds(r0, CHUNK)], wsem[p]
            )

        cvecs = [_iota16(16 * cg) for cg in range(4)]

        def transpose_block(p, nb):
            def tloop(bg, carry):
                for db in range(16):
                    b = bg * 16 + db
                    bvec = jnp.zeros((16,), _i32) + b
                    for cg in range(4):
                        v = plsc.load_gather(tbuf.at[p], [cvecs[cg], bvec])
                        obuf.at[p][b, pl.ds(16 * cg, 16)] = v
                return carry

            lax.fori_loop(0, nb // 16, tloop, 0)

        read_copy(0, 0).start()

        def body(g, carry):
            for pb in range(2):
                t = 2 * g + pb
                read_copy(t, pb).wait()

                @pl.when(t + 1 < K1_PER_W)
                def _():
                    read_copy(t + 1, 1 - pb).start()

                if pb == 0:
                    @pl.when(g > 0)
                    def _():
                        write_copy(t - 2, pb).wait()
                else:
                    @pl.when(g > 0)
                    def _():
                        write_copy(t - 2, pb).wait()
                transpose_block(pb, CHUNK)
                write_copy(t, pb).start()
            return carry

        lax.fori_loop(0, K1_PER_W // 2, body, 0)
        write_copy(K1_PER_W - 2, 0).wait()
        write_copy(K1_PER_W - 1, 1).wait()

        @pl.when(wid < K1_EXTRA)
        def _():
            r0 = (NFULL - K1_EXTRA + wid) * CHUNK
            pltpu.sync_copy(tab_hbm.at[:, pl.ds(r0, CHUNK)], tbuf.at[0])
            transpose_block(0, CHUNK)
            pltpu.sync_copy(obuf.at[0], out_hbm.at[pl.ds(r0, CHUNK)])

        @pl.when(wid == NW - 1)
        def _():
            pltpu.sync_copy(tail_hbm, obuf.at[1, pl.ds(0, TAIL_N)])
            pltpu.sync_copy(
                obuf.at[1, pl.ds(0, TAIL_N)], out_hbm.at[pl.ds(TAIL_R0, TAIL_N)]
            )

    @functools.partial(
        pl.kernel,
        mesh=mesh2,
        compiler_params=params,
        out_type=jax.ShapeDtypeStruct((S, D, B), jnp.float32),
        scratch_types=[
            pltpu.VMEM((S, BCOLS), jnp.int32),
            pltpu.VMEM((GB, CHUNK, DP), jnp.float32),
            pltpu.VMEM((2, D, CHUNK), jnp.float32),
        ] + [pltpu.SemaphoreType.DMA] * (GB + 2),
    )
    def gather_kernel(idx_hbm, table_hbm, out_hbm, idx_v, gbuf, tob, *sems):
        gsem = sems[:GB]
        wsem = sems[GB:]
        wid = lax.axis_index("s") * NC + lax.axis_index("c")
        col0 = wid * BCOLS
        pltpu.sync_copy(idx_hbm.at[:, pl.ds(col0, BCOLS)], idx_v)

        def gather_copy(t, g):
            return pltpu.make_async_copy(
                table_hbm.at[
                    idx_v.at[lax.div(t, 4), pl.ds(lax.rem(t, 4) * CHUNK, CHUNK)]
                ],
                gbuf.at[g],
                gsem[g],
            )

        def write_copy(t, b):
            return pltpu.make_async_copy(
                tob.at[b],
                out_hbm.at[
                    lax.div(t, 4), :, pl.ds(col0 + lax.rem(t, 4) * CHUNK, CHUNK)
                ],
                wsem[b],
            )

        bvecs = [_iota16(16 * bg) for bg in range(8)]

        def transpose_chunk(g, b):
            def tloop(ci, carry):
                for dc in range(4):
                    c = ci * 4 + dc
                    cvec = jnp.zeros((16,), _i32) + c
                    for bg in range(8):
                        v = plsc.load_gather(gbuf.at[g], [bvecs[bg], cvec])
                        tob.at[b][c, pl.ds(16 * bg, 16)] = v
                return carry

            lax.fori_loop(0, D // 4, tloop, 0)

        for t0 in range(GB):
            gather_copy(t0, t0).start()

        def body(q, carry):
            for j in range(GB):
                t = GB * q + j
                b = j % 2
                gather_copy(t, j).wait()
                if j >= 2:
                    write_copy(t - 2, b).wait()
                else:
                    @pl.when(q > 0)
                    def _():
                        write_copy(t - 2, b).wait()
                transpose_chunk(j, b)
                write_copy(t, b).start()

                @pl.when(q < K2_N // GB - 1)
                def _():
                    gather_copy(t + GB, j).start()
            return carry

        lax.fori_loop(0, K2_N // GB, body, 0)
        write_copy(K2_N - 2, 0).wait()
        write_copy(K2_N - 1, 1).wait()

    tail128 = jnp.pad(item_embeddings[TAIL_R0:, :], ((0, 0), (0, DP - D)))
    table128 = retile_kernel(table_t, tail128)
    out_t = gather_kernel(idx_t, table128)
    return jnp.transpose(out_t, (2, 0, 1))
